# baseline (speedup 1.0000x reference)
import functools

import jax
import jax.numpy as jnp
from jax import lax
from jax.experimental import pallas as pl
from jax.experimental.pallas import tpu as pltpu

B, C, HW, N = 16, 768, 32, 1024
HEADS, HD, NTOP = 12, 64, 98
NPAD = 128
PADIDX = 1 << 20
NBG = N - NTOP


def _dg(a, b, dims, batch=((), ())):
    return lax.dot_general(a, b, (dims, batch),
                           preferred_element_type=jnp.float32)


def _iota(shape, dim):
    return lax.broadcasted_iota(jnp.int32, shape, dim)


_PARALLEL = pltpu.CompilerParams(dimension_semantics=("parallel",))


def _stats_body(x_ref, ssum_ref, ssq_ref, scores_ref):
    xb = x_ref[0]
    xx = xb * xb
    scores_ref[...] = jnp.sum(xx, axis=0).reshape(1, 1, N)
    ssum_ref[...] = jnp.sum(xb, axis=1).reshape(1, 1, C)
    ssq_ref[...] = jnp.sum(xx, axis=1).reshape(1, 1, C)


def _stats(x3):
    return pl.pallas_call(
        _stats_body,
        grid=(B,),
        in_specs=[pl.BlockSpec((1, C, N), lambda b: (b, 0, 0))],
        out_specs=[
            pl.BlockSpec((1, 1, C), lambda b: (b, 0, 0)),
            pl.BlockSpec((1, 1, C), lambda b: (b, 0, 0)),
            pl.BlockSpec((1, 1, N), lambda b: (b, 0, 0)),
        ],
        out_shape=[
            jax.ShapeDtypeStruct((B, 1, C), jnp.float32),
            jax.ShapeDtypeStruct((B, 1, C), jnp.float32),
            jax.ShapeDtypeStruct((B, 1, N), jnp.float32),
        ],
        compiler_params=_PARALLEL,
    )(x3)


def _topk_body(scores_ref, slot_ref):
    s = scores_ref[:, 0, :]
    sb = lax.bitcast_convert_type(s, jnp.int32)

    def step(_, carry):
        lo, hi = carry
        mid = lo + ((hi - lo) >> 1)
        cnt = jnp.sum((sb >= mid).astype(jnp.int32), axis=1, keepdims=True)
        ge = cnt >= NTOP
        return jnp.where(ge, mid, lo), jnp.where(ge, hi, mid)

    lo0 = jnp.zeros((B, 1), jnp.int32)
    hi0 = jnp.full((B, 1), jnp.int32(0x7FFFFFFF))
    lo, _ = lax.fori_loop(0, 31, step, (lo0, hi0))
    sel = sb >= lo
    v = sel.astype(jnp.int32)
    sh = 1
    while sh < N:
        v = v + jnp.concatenate(
            [jnp.zeros((B, sh), jnp.int32), v[:, :N - sh]], axis=1)
        sh *= 2
    rank = v - 1
    slot = jnp.where(sel & (rank < NTOP), rank, jnp.int32(PADIDX))
    slot_ref[...] = slot[:, None, :]


def _topk(scores):
    return pl.pallas_call(
        _topk_body,
        out_shape=jax.ShapeDtypeStruct((B, 1, N), jnp.int32),
    )(scores)


def _fold_body(csum_ref, csq_ref, gamma_ref, beta_ref, fc1_w_ref, fc1_b_ref,
               dw_b_ref, fc2_w_ref, fc2_b_ref, proj_w_ref, proj_b_ref,
               w1_ref, m_ref, b1_ref, c_ref):
    inv = jnp.float32(1.0 / (B * N))
    csum = jnp.sum(csum_ref[:, 0, :], axis=0, keepdims=True)
    csq = jnp.sum(csq_ref[:, 0, :], axis=0, keepdims=True)
    mean = csum * inv
    var = csq * inv - mean * mean
    a = gamma_ref[...] * lax.rsqrt(var + 1e-5)
    bvec = beta_ref[...] - mean * a
    fc1_w = fc1_w_ref[...]
    w1_ref[...] = fc1_w * a
    b1_ref[...] = _dg(bvec, fc1_w, ((1,), (1,))) + fc1_b_ref[...]
    proj_w = proj_w_ref[...]
    m = _dg(proj_w, fc2_w_ref[...], ((1,), (0,)))
    m_ref[...] = m
    c_ref[...] = (_dg(dw_b_ref[...], m, ((1,), (1,)))
                  + _dg(fc2_b_ref[...], proj_w, ((1,), (1,)))
                  + proj_b_ref[...])


def _fold(csum, csq, gamma, beta, fc1_w, fc1_b, dw_b, fc2_w, fc2_b,
          proj_w, proj_b):
    return pl.pallas_call(
        _fold_body,
        out_shape=[
            jax.ShapeDtypeStruct((C, C), jnp.float32),
            jax.ShapeDtypeStruct((C, C), jnp.float32),
            jax.ShapeDtypeStruct((1, C), jnp.float32),
            jax.ShapeDtypeStruct((1, C), jnp.float32),
        ],
    )(csum, csq, gamma, beta, fc1_w, fc1_b, dw_b, fc2_w, fc2_b,
      proj_w, proj_b)


def _gelu(h):
    return 0.5 * h * (1.0 + lax.erf(h * jnp.float32(0.7071067811865476)))


def _main_body(x_ref, idx_ref, w1_ref, b1_ref, m_ref, c_ref, wqkv_ref,
               bqkv_ref, proj_w_ref, dwtap_ref, icol_ref, irow_ref, c98_ref,
               r98_ref, kmask_ref, wm0_ref, wm1_ref, out_ref):
    for bi in range(2):
        _main_one(bi, x_ref, idx_ref, w1_ref, b1_ref, m_ref, c_ref,
                  wqkv_ref, bqkv_ref, proj_w_ref, dwtap_ref, icol_ref,
                  irow_ref, c98_ref, r98_ref, kmask_ref, wm0_ref, wm1_ref,
                  out_ref)


def _main_one(bi, x_ref, idx_ref, w1_ref, b1_ref, m_ref, c_ref, wqkv_ref,
              bqkv_ref, proj_w_ref, dwtap_ref, icol_ref, irow_ref, c98_ref,
              r98_ref, kmask_ref, wm0_ref, wm1_ref, out_ref):
    xb = x_ref[bi]
    xb16 = xb.astype(jnp.bfloat16)
    slotrow = idx_ref[bi, 0, :]

    oneh = (slotrow[:, None] == icol_ref[...]).astype(jnp.float32)
    topmask = (slotrow[:, None] < NTOP).astype(jnp.float32)
    sg = oneh + ((1.0 - topmask) * jnp.float32(1.0 / NBG)) * c98_ref[...]
    tok = _dg(xb16, sg.astype(jnp.bfloat16), ((1,), (0,)))

    qkvT = (_dg(wqkv_ref[...], tok.astype(jnp.bfloat16), ((1,), (0,)))
            + bqkv_ref[...])
    qkv4 = qkvT.reshape(3, HEADS, HD, NPAD)
    kmask = kmask_ref[...]
    q, k, v = qkv4[0], qkv4[1], qkv4[2]
    q16 = q.astype(jnp.bfloat16)
    k16 = k.astype(jnp.bfloat16)
    lg = (_dg(q16, k16, ((1,), (1,)), ((0,), (0,))) * jnp.float32(HD ** -0.5)
          + kmask[None])
    mx = jnp.max(lg, axis=2, keepdims=True)
    e = jnp.exp(lg - mx)
    p = e / jnp.sum(e, axis=2, keepdims=True)
    res = _dg(v.astype(jnp.bfloat16), p.astype(jnp.bfloat16),
              ((2,), (2,)), ((0,), (0,)))
    attn_out = res.reshape(C, NPAD)
    proj_top = _dg(proj_w_ref[...], attn_out.astype(jnp.bfloat16),
                   ((1,), (0,)))

    onehT = (irow_ref[...] == slotrow[None, :]).astype(jnp.float32)
    bgrow = 1.0 - (slotrow[None, :] < NTOP).astype(jnp.float32)
    g = onehT + r98_ref[...] * bgrow
    sal = _dg(proj_top.astype(jnp.bfloat16), g.astype(jnp.bfloat16),
              ((1,), (0,)))

    h1f = _gelu(_dg(w1_ref[...], xb16, ((1,), (0,))) + b1_ref[...])
    h1 = h1f.astype(jnp.bfloat16)
    wm0 = wm0_ref[...]
    wm1 = wm1_ref[...]
    dwtap = dwtap_ref[...]
    acc = jnp.zeros((C, N), jnp.bfloat16)
    t = 0
    for dy in (-1, 0, 1):
        for dx in (-1, 0, 1):
            s = dy * HW + dx
            if s > 0:
                sh = jnp.concatenate(
                    [h1[:, s:], jnp.zeros((C, s), jnp.bfloat16)], axis=1)
            elif s < 0:
                sh = jnp.concatenate(
                    [jnp.zeros((C, -s), jnp.bfloat16), h1[:, :N + s]],
                    axis=1)
            else:
                sh = h1
            if dx == 1:
                sh = sh * wm1
            elif dx == -1:
                sh = sh * wm0
            acc = acc + dwtap[:, t:t + 1] * sh
            t += 1
    dense = _dg(m_ref[...], acc, ((1,), (0,)))
    out_ref[bi] = dense + c_ref[...] + sal


def _main(x3, slot, w1, b1c, m, cc, wqkv, bqkvc, proj_w, dwtap,
          icol, irow, c98, r98, kmask, wm0, wm1):
    wspec = pl.BlockSpec((C, C), lambda b: (0, 0))
    cspec = pl.BlockSpec((C, 1), lambda b: (0, 0))
    rspec = pl.BlockSpec((1, N), lambda b: (0, 0))
    return pl.pallas_call(
        _main_body,
        grid=(B // 2,),
        in_specs=[
            pl.BlockSpec((2, C, N), lambda b: (b, 0, 0)),
            pl.BlockSpec((2, 1, N), lambda b: (b, 0, 0)),
            wspec, cspec, wspec, cspec,
            pl.BlockSpec((3 * C, C), lambda b: (0, 0)),
            pl.BlockSpec((3 * C, 1), lambda b: (0, 0)),
            wspec,
            pl.BlockSpec((C, 9), lambda b: (0, 0)),
            pl.BlockSpec((1, NPAD), lambda b: (0, 0)),
            pl.BlockSpec((NPAD, 1), lambda b: (0, 0)),
            pl.BlockSpec((1, NPAD), lambda b: (0, 0)),
            pl.BlockSpec((NPAD, 1), lambda b: (0, 0)),
            pl.BlockSpec((1, NPAD), lambda b: (0, 0)),
            rspec, rspec,
        ],
        out_specs=pl.BlockSpec((2, C, N), lambda b: (b, 0, 0)),
        out_shape=jax.ShapeDtypeStruct((B, C, N), jnp.float32),
        compiler_params=_PARALLEL,
    )(x3, slot, w1, b1c, m, cc, wqkv, bqkvc, proj_w, dwtap,
      icol, irow, c98, r98, kmask, wm0, wm1)


def kernel(x, Wqkv, bqkv, gamma, beta, fc1_w, fc1_b, dw_w, dw_b, fc2_w,
           fc2_b, proj_w, proj_b):
    x3 = x.reshape(B, C, N)
    ssum, ssq, scores = _stats(x3)
    slot = _topk(scores)
    w1, m, b1row, crow = _fold(
        ssum, ssq, gamma.reshape(1, C),
        beta.reshape(1, C), fc1_w, fc1_b.reshape(1, C), dw_b.reshape(1, C),
        fc2_w, fc2_b.reshape(1, C), proj_w, proj_b.reshape(1, C))

    icol = lax.broadcasted_iota(jnp.int32, (1, NPAD), 1)
    irow = lax.broadcasted_iota(jnp.int32, (NPAD, 1), 0)
    c98 = (lax.broadcasted_iota(jnp.int32, (1, NPAD), 1) == NTOP
           ).astype(jnp.float32)
    r98 = (lax.broadcasted_iota(jnp.int32, (NPAD, 1), 0) == NTOP
           ).astype(jnp.float32)
    kmask = jnp.where(lax.broadcasted_iota(jnp.int32, (1, NPAD), 1) < NTOP + 1,
                      0.0, -1e30).astype(jnp.float32)
    wcoord = lax.broadcasted_iota(jnp.int32, (1, N), 1) % HW
    wm0 = (wcoord != 0).astype(jnp.bfloat16)
    wm1 = (wcoord != HW - 1).astype(jnp.bfloat16)

    bf = jnp.bfloat16
    out3 = _main(x3, slot, w1.astype(bf), b1row.reshape(C, 1), m.astype(bf),
                 crow.reshape(C, 1), Wqkv.astype(bf),
                 bqkv.reshape(3 * C, 1), proj_w.astype(bf),
                 dw_w.reshape(C, 9).astype(bf),
                 icol, irow, c98, r98, kmask, wm0, wm1)
    return out3.reshape(B, C, HW, HW)


# speedup vs baseline: 1.0330x; 1.0330x over previous
import functools

import jax
import jax.numpy as jnp
from jax import lax
from jax.experimental import pallas as pl
from jax.experimental.pallas import tpu as pltpu

B, C, HW, N = 16, 768, 32, 1024
HEADS, HD, NTOP = 12, 64, 98
NPAD = 128
PADIDX = 1 << 20
NBG = N - NTOP


def _dg(a, b, dims, batch=((), ())):
    return lax.dot_general(a, b, (dims, batch),
                           preferred_element_type=jnp.float32)


def _iota(shape, dim):
    return lax.broadcasted_iota(jnp.int32, shape, dim)


_PARALLEL = pltpu.CompilerParams(dimension_semantics=("parallel",))


def _stats_body(x_ref, ssum_ref, ssq_ref, scores_ref):
    xb = x_ref[0]
    xx = xb * xb
    scores_ref[...] = jnp.sum(xx, axis=0).reshape(1, 1, N)
    ssum_ref[...] = jnp.sum(xb, axis=1).reshape(1, 1, C)
    ssq_ref[...] = jnp.sum(xx, axis=1).reshape(1, 1, C)


def _stats(x3):
    return pl.pallas_call(
        _stats_body,
        grid=(B,),
        in_specs=[pl.BlockSpec((1, C, N), lambda b: (b, 0, 0))],
        out_specs=[
            pl.BlockSpec((1, 1, C), lambda b: (b, 0, 0)),
            pl.BlockSpec((1, 1, C), lambda b: (b, 0, 0)),
            pl.BlockSpec((1, 1, N), lambda b: (b, 0, 0)),
        ],
        out_shape=[
            jax.ShapeDtypeStruct((B, 1, C), jnp.float32),
            jax.ShapeDtypeStruct((B, 1, C), jnp.float32),
            jax.ShapeDtypeStruct((B, 1, N), jnp.float32),
        ],
        compiler_params=_PARALLEL,
    )(x3)


def _topk_body(scores_ref, slot_ref):
    s = scores_ref[:, 0, :]
    sb = lax.bitcast_convert_type(s, jnp.int32)

    def step(_, carry):
        lo, hi = carry
        mid = lo + ((hi - lo) >> 1)
        cnt = jnp.sum((sb >= mid).astype(jnp.int32), axis=1, keepdims=True)
        ge = cnt >= NTOP
        return jnp.where(ge, mid, lo), jnp.where(ge, hi, mid)

    lo0 = jnp.zeros((B, 1), jnp.int32)
    hi0 = jnp.full((B, 1), jnp.int32(0x7FFFFFFF))
    lo, _ = lax.fori_loop(0, 31, step, (lo0, hi0))
    sel = sb >= lo
    v = sel.astype(jnp.int32)
    sh = 1
    while sh < N:
        v = v + jnp.concatenate(
            [jnp.zeros((B, sh), jnp.int32), v[:, :N - sh]], axis=1)
        sh *= 2
    rank = v - 1
    slot = jnp.where(sel & (rank < NTOP), rank, jnp.int32(PADIDX))
    slot_ref[...] = slot[:, None, :]


def _topk(scores):
    return pl.pallas_call(
        _topk_body,
        out_shape=jax.ShapeDtypeStruct((B, 1, N), jnp.int32),
    )(scores)


def _fold_body(csum_ref, csq_ref, gamma_ref, beta_ref, fc1_w_ref, fc1_b_ref,
               dw_b_ref, fc2_w_ref, fc2_b_ref, proj_w_ref, proj_b_ref,
               w1_ref, m_ref, b1_ref, c_ref):
    inv = jnp.float32(1.0 / (B * N))
    csum = jnp.sum(csum_ref[:, 0, :], axis=0, keepdims=True)
    csq = jnp.sum(csq_ref[:, 0, :], axis=0, keepdims=True)
    mean = csum * inv
    var = csq * inv - mean * mean
    a = gamma_ref[...] * lax.rsqrt(var + 1e-5)
    bvec = beta_ref[...] - mean * a
    fc1_w = fc1_w_ref[...]
    w1_ref[...] = fc1_w * a
    b1_ref[...] = _dg(bvec, fc1_w, ((1,), (1,))) + fc1_b_ref[...]
    proj_w = proj_w_ref[...]
    m = _dg(proj_w, fc2_w_ref[...], ((1,), (0,)))
    m_ref[...] = m
    c_ref[...] = (_dg(dw_b_ref[...], m, ((1,), (1,)))
                  + _dg(fc2_b_ref[...], proj_w, ((1,), (1,)))
                  + proj_b_ref[...])


def _fold(csum, csq, gamma, beta, fc1_w, fc1_b, dw_b, fc2_w, fc2_b,
          proj_w, proj_b):
    return pl.pallas_call(
        _fold_body,
        out_shape=[
            jax.ShapeDtypeStruct((C, C), jnp.float32),
            jax.ShapeDtypeStruct((C, C), jnp.float32),
            jax.ShapeDtypeStruct((1, C), jnp.float32),
            jax.ShapeDtypeStruct((1, C), jnp.float32),
        ],
    )(csum, csq, gamma, beta, fc1_w, fc1_b, dw_b, fc2_w, fc2_b,
      proj_w, proj_b)


def _gelu(h):
    return 0.5 * h * (1.0 + lax.erf(h * jnp.float32(0.7071067811865476)))


def _main_body(x_ref, idx_ref, w1_ref, b1_ref, m_ref, c_ref, wqkv_ref,
               bqkv_ref, proj_w_ref, dwtap_ref, icol_ref, irow_ref, c98_ref,
               r98_ref, kmask_ref, wm0_ref, wm1_ref, out_ref):
    xb = x_ref[0]
    xb16 = xb.astype(jnp.bfloat16)
    slotrow = idx_ref[0, 0, :]

    oneh = (slotrow[:, None] == icol_ref[...]).astype(jnp.float32)
    topmask = (slotrow[:, None] < NTOP).astype(jnp.float32)
    sg = oneh + ((1.0 - topmask) * jnp.float32(1.0 / NBG)) * c98_ref[...]
    tok = _dg(xb16, sg.astype(jnp.bfloat16), ((1,), (0,)))

    qkvT = (_dg(wqkv_ref[...], tok.astype(jnp.bfloat16), ((1,), (0,)))
            + bqkv_ref[...])
    qkv4 = qkvT.reshape(3, HEADS, HD, NPAD)
    kmask = kmask_ref[...]
    q, k, v = qkv4[0], qkv4[1], qkv4[2]
    q16 = q.astype(jnp.bfloat16)
    k16 = k.astype(jnp.bfloat16)
    lg = (_dg(q16, k16, ((1,), (1,)), ((0,), (0,))) * jnp.float32(HD ** -0.5)
          + kmask[None])
    mx = jnp.max(lg, axis=2, keepdims=True)
    e = jnp.exp(lg - mx)
    p = e / jnp.sum(e, axis=2, keepdims=True)
    res = _dg(v.astype(jnp.bfloat16), p.astype(jnp.bfloat16),
              ((2,), (2,)), ((0,), (0,)))
    attn_out = res.reshape(C, NPAD)
    proj_top = _dg(proj_w_ref[...], attn_out.astype(jnp.bfloat16),
                   ((1,), (0,)))

    onehT = (irow_ref[...] == slotrow[None, :]).astype(jnp.float32)
    bgrow = 1.0 - (slotrow[None, :] < NTOP).astype(jnp.float32)
    g = onehT + r98_ref[...] * bgrow
    sal = _dg(proj_top.astype(jnp.bfloat16), g.astype(jnp.bfloat16),
              ((1,), (0,)))

    h1f = _gelu(_dg(w1_ref[...], xb16, ((1,), (0,))) + b1_ref[...])
    h1 = h1f.astype(jnp.bfloat16)
    wm0 = wm0_ref[...]
    wm1 = wm1_ref[...]
    dwtap = dwtap_ref[...]
    acc = jnp.zeros((C, N), jnp.bfloat16)
    t = 0
    for dy in (-1, 0, 1):
        for dx in (-1, 0, 1):
            s = dy * HW + dx
            if s > 0:
                sh = jnp.concatenate(
                    [h1[:, s:], jnp.zeros((C, s), jnp.bfloat16)], axis=1)
            elif s < 0:
                sh = jnp.concatenate(
                    [jnp.zeros((C, -s), jnp.bfloat16), h1[:, :N + s]],
                    axis=1)
            else:
                sh = h1
            if dx == 1:
                sh = sh * wm1
            elif dx == -1:
                sh = sh * wm0
            acc = acc + dwtap[:, t:t + 1] * sh
            t += 1
    dense = _dg(m_ref[...], acc, ((1,), (0,)))
    out_ref[...] = (dense + c_ref[...] + sal)[None]


def _main(x3, slot, w1, b1c, m, cc, wqkv, bqkvc, proj_w, dwtap,
          icol, irow, c98, r98, kmask, wm0, wm1):
    wspec = pl.BlockSpec((C, C), lambda b: (0, 0))
    cspec = pl.BlockSpec((C, 1), lambda b: (0, 0))
    rspec = pl.BlockSpec((1, N), lambda b: (0, 0))
    return pl.pallas_call(
        _main_body,
        grid=(B,),
        in_specs=[
            pl.BlockSpec((1, C, N), lambda b: (b, 0, 0)),
            pl.BlockSpec((1, 1, N), lambda b: (b, 0, 0)),
            wspec, cspec, wspec, cspec,
            pl.BlockSpec((3 * C, C), lambda b: (0, 0)),
            pl.BlockSpec((3 * C, 1), lambda b: (0, 0)),
            wspec,
            pl.BlockSpec((C, 9), lambda b: (0, 0)),
            pl.BlockSpec((1, NPAD), lambda b: (0, 0)),
            pl.BlockSpec((NPAD, 1), lambda b: (0, 0)),
            pl.BlockSpec((1, NPAD), lambda b: (0, 0)),
            pl.BlockSpec((NPAD, 1), lambda b: (0, 0)),
            pl.BlockSpec((1, NPAD), lambda b: (0, 0)),
            rspec, rspec,
        ],
        out_specs=pl.BlockSpec((1, C, N), lambda b: (b, 0, 0)),
        out_shape=jax.ShapeDtypeStruct((B, C, N), jnp.float32),
        compiler_params=_PARALLEL,
    )(x3, slot, w1, b1c, m, cc, wqkv, bqkvc, proj_w, dwtap,
      icol, irow, c98, r98, kmask, wm0, wm1)


def kernel(x, Wqkv, bqkv, gamma, beta, fc1_w, fc1_b, dw_w, dw_b, fc2_w,
           fc2_b, proj_w, proj_b):
    x3 = x.reshape(B, C, N)
    ssum, ssq, scores = _stats(x3)
    slot = _topk(scores)
    w1, m, b1row, crow = _fold(
        ssum, ssq, gamma.reshape(1, C),
        beta.reshape(1, C), fc1_w, fc1_b.reshape(1, C), dw_b.reshape(1, C),
        fc2_w, fc2_b.reshape(1, C), proj_w, proj_b.reshape(1, C))

    icol = lax.broadcasted_iota(jnp.int32, (1, NPAD), 1)
    irow = lax.broadcasted_iota(jnp.int32, (NPAD, 1), 0)
    c98 = (lax.broadcasted_iota(jnp.int32, (1, NPAD), 1) == NTOP
           ).astype(jnp.float32)
    r98 = (lax.broadcasted_iota(jnp.int32, (NPAD, 1), 0) == NTOP
           ).astype(jnp.float32)
    kmask = jnp.where(lax.broadcasted_iota(jnp.int32, (1, NPAD), 1) < NTOP + 1,
                      0.0, -1e30).astype(jnp.float32)
    wcoord = lax.broadcasted_iota(jnp.int32, (1, N), 1) % HW
    wm0 = (wcoord != 0).astype(jnp.bfloat16)
    wm1 = (wcoord != HW - 1).astype(jnp.bfloat16)

    bf = jnp.bfloat16
    out3 = _main(x3, slot, w1.astype(bf), b1row.reshape(C, 1), m.astype(bf),
                 crow.reshape(C, 1), Wqkv.astype(bf),
                 bqkv.reshape(3 * C, 1), proj_w.astype(bf),
                 dw_w.reshape(C, 9).astype(bf),
                 icol, irow, c98, r98, kmask, wm0, wm1)
    return out3.reshape(B, C, HW, HW)


# speedup vs baseline: 1.0356x; 1.0025x over previous
import jax
import jax.numpy as jnp
from jax import lax
from jax.experimental import pallas as pl
from jax.experimental.pallas import tpu as pltpu

B, C, HW, N = 16, 768, 32, 1024
HEADS, HD, NTOP = 12, 64, 98
NPAD = 128
PADIDX = 1 << 20
NBG = N - NTOP


def _dg(a, b, dims, batch=((), ())):
    return lax.dot_general(a, b, (dims, batch),
                           preferred_element_type=jnp.float32)


_PARALLEL = pltpu.CompilerParams(dimension_semantics=("parallel",))


def _stats_body(x_ref, ssum_ref, ssq_ref, scores_ref):
    xb = x_ref[0]
    xx = xb * xb
    scores_ref[...] = jnp.sum(xx, axis=0).reshape(1, 1, N)
    ssum_ref[...] = jnp.sum(xb, axis=1).reshape(1, 1, C)
    ssq_ref[...] = jnp.sum(xx, axis=1).reshape(1, 1, C)


def _stats(x3):
    return pl.pallas_call(
        _stats_body,
        grid=(B,),
        in_specs=[pl.BlockSpec((1, C, N), lambda b: (b, 0, 0))],
        out_specs=[
            pl.BlockSpec((1, 1, C), lambda b: (b, 0, 0)),
            pl.BlockSpec((1, 1, C), lambda b: (b, 0, 0)),
            pl.BlockSpec((1, 1, N), lambda b: (b, 0, 0)),
        ],
        out_shape=[
            jax.ShapeDtypeStruct((B, 1, C), jnp.float32),
            jax.ShapeDtypeStruct((B, 1, C), jnp.float32),
            jax.ShapeDtypeStruct((B, 1, N), jnp.float32),
        ],
        compiler_params=_PARALLEL,
    )(x3)


def _topk_body(scores_ref, slot_ref):
    s = scores_ref[:, 0, :]
    sb = lax.bitcast_convert_type(s, jnp.int32)

    def step(_, carry):
        lo, hi = carry
        mid = lo + ((hi - lo) >> 1)
        cnt = jnp.sum((sb >= mid).astype(jnp.int32), axis=1, keepdims=True)
        ge = cnt >= NTOP
        return jnp.where(ge, mid, lo), jnp.where(ge, hi, mid)

    lo0 = jnp.zeros((B, 1), jnp.int32)
    hi0 = jnp.full((B, 1), jnp.int32(0x7FFFFFFF))
    lo, _ = lax.fori_loop(0, 31, step, (lo0, hi0))
    sel = sb >= lo
    v = sel.astype(jnp.int32)
    sh = 1
    while sh < N:
        v = v + jnp.concatenate(
            [jnp.zeros((B, sh), jnp.int32), v[:, :N - sh]], axis=1)
        sh *= 2
    rank = v - 1
    slot = jnp.where(sel & (rank < NTOP), rank, jnp.int32(PADIDX))
    slot_ref[...] = slot[:, None, :]


def _topk(scores):
    return pl.pallas_call(
        _topk_body,
        out_shape=jax.ShapeDtypeStruct((B, 1, N), jnp.int32),
    )(scores)


def _fold_body(csum_ref, csq_ref, gamma_ref, beta_ref, fc1_w_ref, fc1_b_ref,
               dw_b_ref, fc2_w_ref, fc2_b_ref, proj_w_ref, proj_b_ref,
               w1_ref, m_ref, b1_ref, c_ref):
    inv = jnp.float32(1.0 / (B * N))
    csum = jnp.sum(csum_ref[:, 0, :], axis=0, keepdims=True)
    csq = jnp.sum(csq_ref[:, 0, :], axis=0, keepdims=True)
    mean = csum * inv
    var = csq * inv - mean * mean
    a = gamma_ref[...] * lax.rsqrt(var + 1e-5)
    bvec = beta_ref[...] - mean * a
    fc1_w = fc1_w_ref[...]
    w1_ref[...] = fc1_w * a
    b1_ref[...] = _dg(bvec, fc1_w, ((1,), (1,))) + fc1_b_ref[...]
    proj_w = proj_w_ref[...]
    m = _dg(proj_w, fc2_w_ref[...], ((1,), (0,)))
    m_ref[...] = m
    c_ref[...] = (_dg(dw_b_ref[...], m, ((1,), (1,)))
                  + _dg(fc2_b_ref[...], proj_w, ((1,), (1,)))
                  + proj_b_ref[...])


def _fold(csum, csq, gamma, beta, fc1_w, fc1_b, dw_b, fc2_w, fc2_b,
          proj_w, proj_b):
    return pl.pallas_call(
        _fold_body,
        out_shape=[
            jax.ShapeDtypeStruct((C, C), jnp.float32),
            jax.ShapeDtypeStruct((C, C), jnp.float32),
            jax.ShapeDtypeStruct((1, C), jnp.float32),
            jax.ShapeDtypeStruct((1, C), jnp.float32),
        ],
    )(csum, csq, gamma, beta, fc1_w, fc1_b, dw_b, fc2_w, fc2_b,
      proj_w, proj_b)


def _gelu(h):
    return 0.5 * h * (1.0 + lax.erf(h * jnp.float32(0.7071067811865476)))


def _main_body(x_ref, idx_ref, w1_ref, b1_ref, m_ref, c_ref, wqkv_ref,
               bqkv_ref, proj_w_ref, dwtap_ref, icol_ref, irow_ref, c98_ref,
               r98_ref, kmask_ref, wm0_ref, wm1_ref, out_ref):
    xb = x_ref[0]
    xb16 = xb.astype(jnp.bfloat16)
    slotrow = idx_ref[0, 0, :]

    oneh = (slotrow[:, None] == icol_ref[...]).astype(jnp.float32)
    topmask = (slotrow[:, None] < NTOP).astype(jnp.float32)
    sg = oneh + ((1.0 - topmask) * jnp.float32(1.0 / NBG)) * c98_ref[...]
    tok = _dg(xb16, sg.astype(jnp.bfloat16), ((1,), (0,)))

    qkvT = (_dg(wqkv_ref[...], tok.astype(jnp.bfloat16), ((1,), (0,)))
            + bqkv_ref[...])
    qkv4 = qkvT.reshape(3, HEADS, HD, NPAD)
    kmask = kmask_ref[...]
    q, k, v = qkv4[0], qkv4[1], qkv4[2]
    q16 = q.astype(jnp.bfloat16)
    k16 = k.astype(jnp.bfloat16)
    lg = (_dg(q16, k16, ((1,), (1,)), ((0,), (0,))) * jnp.float32(HD ** -0.5)
          + kmask[None])
    mx = jnp.max(lg, axis=2, keepdims=True)
    e = jnp.exp(lg - mx)
    p = e / jnp.sum(e, axis=2, keepdims=True)
    res = _dg(v.astype(jnp.bfloat16), p.astype(jnp.bfloat16),
              ((2,), (2,)), ((0,), (0,)))
    attn_out = res.reshape(C, NPAD)
    proj_top = _dg(proj_w_ref[...], attn_out.astype(jnp.bfloat16),
                   ((1,), (0,)))

    onehT = (irow_ref[...] == slotrow[None, :]).astype(jnp.float32)
    bgrow = 1.0 - (slotrow[None, :] < NTOP).astype(jnp.float32)
    g = onehT + r98_ref[...] * bgrow
    sal = _dg(proj_top.astype(jnp.bfloat16), g.astype(jnp.bfloat16),
              ((1,), (0,)))

    h1f = _gelu(_dg(w1_ref[...], xb16, ((1,), (0,))) + b1_ref[...])
    h1 = h1f.astype(jnp.bfloat16)
    wm0 = wm0_ref[...]
    wm1 = wm1_ref[...]
    dwtap = dwtap_ref[...]
    acc = jnp.zeros((C, N), jnp.bfloat16)
    t = 0
    for dy in (-1, 0, 1):
        for dx in (-1, 0, 1):
            s = dy * HW + dx
            if s > 0:
                sh = jnp.concatenate(
                    [h1[:, s:], jnp.zeros((C, s), jnp.bfloat16)], axis=1)
            elif s < 0:
                sh = jnp.concatenate(
                    [jnp.zeros((C, -s), jnp.bfloat16), h1[:, :N + s]],
                    axis=1)
            else:
                sh = h1
            if dx == 1:
                sh = sh * wm1
            elif dx == -1:
                sh = sh * wm0
            acc = acc + dwtap[:, t:t + 1] * sh
            t += 1
    dense = _dg(m_ref[...], acc, ((1,), (0,)))
    out_ref[...] = (dense + c_ref[...] + sal)[None]


def _main(x3, slot, w1, b1c, m, cc, wqkv, bqkvc, proj_w, dwtap,
          icol, irow, c98, r98, kmask, wm0, wm1):
    wspec = pl.BlockSpec((C, C), lambda b: (0, 0))
    cspec = pl.BlockSpec((C, 1), lambda b: (0, 0))
    rspec = pl.BlockSpec((1, N), lambda b: (0, 0))
    return pl.pallas_call(
        _main_body,
        grid=(B,),
        in_specs=[
            pl.BlockSpec((1, C, N), lambda b: (b, 0, 0)),
            pl.BlockSpec((1, 1, N), lambda b: (b, 0, 0)),
            wspec, cspec, wspec, cspec,
            pl.BlockSpec((3 * C, C), lambda b: (0, 0)),
            pl.BlockSpec((3 * C, 1), lambda b: (0, 0)),
            wspec,
            pl.BlockSpec((C, 9), lambda b: (0, 0)),
            pl.BlockSpec((1, NPAD), lambda b: (0, 0)),
            pl.BlockSpec((NPAD, 1), lambda b: (0, 0)),
            pl.BlockSpec((1, NPAD), lambda b: (0, 0)),
            pl.BlockSpec((NPAD, 1), lambda b: (0, 0)),
            pl.BlockSpec((1, NPAD), lambda b: (0, 0)),
            rspec, rspec,
        ],
        out_specs=pl.BlockSpec((1, C, N), lambda b: (b, 0, 0)),
        out_shape=jax.ShapeDtypeStruct((B, C, N), jnp.float32),
        compiler_params=_PARALLEL,
    )(x3, slot, w1, b1c, m, cc, wqkv, bqkvc, proj_w, dwtap,
      icol, irow, c98, r98, kmask, wm0, wm1)


def kernel(x, Wqkv, bqkv, gamma, beta, fc1_w, fc1_b, dw_w, dw_b, fc2_w,
           fc2_b, proj_w, proj_b):
    x3 = x.reshape(B, C, N)
    ssum, ssq, scores = _stats(x3)
    slot = _topk(scores)
    w1, m, b1row, crow = _fold(
        ssum, ssq, gamma.reshape(1, C),
        beta.reshape(1, C), fc1_w, fc1_b.reshape(1, C), dw_b.reshape(1, C),
        fc2_w, fc2_b.reshape(1, C), proj_w, proj_b.reshape(1, C))

    icol = lax.broadcasted_iota(jnp.int32, (1, NPAD), 1)
    irow = lax.broadcasted_iota(jnp.int32, (NPAD, 1), 0)
    c98 = (lax.broadcasted_iota(jnp.int32, (1, NPAD), 1) == NTOP
           ).astype(jnp.float32)
    r98 = (lax.broadcasted_iota(jnp.int32, (NPAD, 1), 0) == NTOP
           ).astype(jnp.float32)
    kmask = jnp.where(lax.broadcasted_iota(jnp.int32, (1, NPAD), 1) < NTOP + 1,
                      0.0, -1e30).astype(jnp.float32)
    wcoord = lax.broadcasted_iota(jnp.int32, (1, N), 1) % HW
    wm0 = (wcoord != 0).astype(jnp.bfloat16)
    wm1 = (wcoord != HW - 1).astype(jnp.bfloat16)

    bf = jnp.bfloat16
    out3 = _main(x3, slot, w1.astype(bf), b1row.reshape(C, 1), m.astype(bf),
                 crow.reshape(C, 1), Wqkv.astype(bf),
                 bqkv.reshape(3 * C, 1), proj_w.astype(bf),
                 dw_w.reshape(C, 9).astype(bf),
                 icol, irow, c98, r98, kmask, wm0, wm1)
    return out3.reshape(B, C, HW, HW)
